# baseline (speedup 1.0000x reference)
import jax
import jax.numpy as jnp
from jax.experimental import pallas as pl
from jax.experimental.pallas import tpu as pltpu
from jax.experimental.pallas import tpu_sc as plsc

_MLP_TILE = 1024
_NC, _NS = 2, 16
_NW = _NC * _NS
_SB = 128


def _sc_gather(table, x_pad, win, vocab):
    batch, win_pad = x_pad.shape
    emb = table.shape[1]
    kd = win // 4
    assert win % 4 == 0 and 4 * emb == 128
    npc = _SB * win
    assert npc % 128 == 0
    streams = npc // 128
    kblk = npc // kd
    b_per_w = batch // _NW
    chunks = b_per_w // _SB
    mesh = plsc.VectorSubcoreMesh(core_axis_name="core", subcore_axis_name="subcore")

    @pl.kernel(
        out_type=jax.ShapeDtypeStruct((kd * batch, 4 * emb), table.dtype),
        mesh=mesh,
        scratch_types=[
            pltpu.VMEM((_SB, win_pad), jnp.int32),
            pltpu.VMEM((npc,), jnp.int32),
            pltpu.VMEM((npc, emb), table.dtype),
            pltpu.SemaphoreType.DMA,
        ],
        compiler_params=pltpu.CompilerParams(
            use_tc_tiling_on_sc=False, needs_layout_passes=False
        ),
    )
    def gather_kernel(tab_hbm, i_hbm, o_hbm, idx_v, idxp_v, rows_v, sem):
        wid = jax.lax.axis_index("subcore") * _NC + jax.lax.axis_index("core")
        b0 = wid * b_per_w
        lane = jax.lax.broadcasted_iota(jnp.int32, (16,), 0)

        @pl.loop(0, chunks)
        def _(c):
            b = b0 + c * _SB
            pltpu.sync_copy(i_hbm.at[pl.ds(b, _SB)], idx_v)
            @pl.loop(0, win)
            def _(w):
                for u in range(_SB // 16):
                    rows = 16 * u + lane
                    cols = jnp.full((16,), 0, jnp.int32) + w
                    vals = plsc.load_gather(idx_v, [rows, cols])
                    v4 = vocab // 4
                    quarter = (
                        (vals >= v4).astype(jnp.int32)
                        + (vals >= 2 * v4).astype(jnp.int32)
                        + (vals >= 3 * v4).astype(jnp.int32)
                    )
                    idxp_v[pl.ds(w * _SB + 16 * u, 16)] = (
                        4 * vals - quarter * (vocab - 1)
                    )
            copies = [
                pltpu.async_copy(
                    tab_hbm.at[idxp_v.at[pl.ds(j * 128, 128)]],
                    rows_v.at[pl.ds(j * 128, 128)],
                    sem,
                )
                for j in range(streams)
            ]
            for cp in copies:
                cp.wait()
            for w in range(win):
                k, c4 = divmod(w, 4)
                pltpu.sync_copy(
                    rows_v.at[pl.ds(w * _SB, _SB)],
                    o_hbm.at[pl.ds(k * batch + b, _SB), pl.ds(c4 * emb, emb)],
                )

    return gather_kernel(table, x_pad)


def _compact_body(t0_ref, t1_ref, t2_ref, t3_ref, o_ref):
    emb = t0_ref.shape[1]
    for c, t_ref in enumerate((t0_ref, t1_ref, t2_ref, t3_ref)):
        o_ref[:, pl.ds(c * emb, emb)] = t_ref[...]


def _tc_compact(table):
    v, emb = table.shape
    v4 = v // 4
    rows = 5000
    assert v4 % rows == 0
    nb = v4 // rows
    spec = lambda c: pl.BlockSpec((rows, emb), lambda i, c=c: (c * nb + i, 0))
    return pl.pallas_call(
        _compact_body,
        grid=(nb,),
        in_specs=[spec(0), spec(1), spec(2), spec(3)],
        out_specs=pl.BlockSpec((rows, 4 * emb), lambda i: (i, 0)),
        out_shape=jax.ShapeDtypeStruct((v4, 4 * emb), jnp.float32),
    )(table, table, table, table)


def _mlp_body(h_ref, w1_ref, b1_ref, w2_ref, b2_ref, w3_ref, b3_ref, o_ref):
    kd = h_ref.shape[0]
    z = b1_ref[...] + jnp.dot(h_ref[0], w1_ref[0], preferred_element_type=jnp.float32)
    for k in range(1, kd):
        z = z + jnp.dot(h_ref[k], w1_ref[k], preferred_element_type=jnp.float32)
    z = jnp.maximum(z, 0.0)
    z = jnp.dot(z, w2_ref[...], preferred_element_type=jnp.float32) + b2_ref[...]
    z = jnp.maximum(z, 0.0)
    o_ref[...] = jnp.dot(z, w3_ref[...], preferred_element_type=jnp.float32) + b3_ref[...]


def _tc_mlp(h5, W1, b1, W2, b2, W3, b3):
    kd, batch, lane = h5.shape
    h1, h2, ncls = W1.shape[1], W2.shape[1], W3.shape[1]
    tile = min(_MLP_TILE, batch)
    grid = (batch // tile,)
    full = lambda shape: pl.BlockSpec(shape, lambda i: (0,) * len(shape))
    return pl.pallas_call(
        _mlp_body,
        grid=grid,
        in_specs=[
            pl.BlockSpec((kd, tile, lane), lambda i: (0, i, 0)),
            full((kd, lane, h1)),
            full((1, h1)),
            full((h1, h2)),
            full((1, h2)),
            full((h2, ncls)),
            full((1, ncls)),
        ],
        out_specs=pl.BlockSpec((tile, ncls), lambda i: (i, 0)),
        out_shape=jax.ShapeDtypeStruct((batch, ncls), jnp.float32),
    )(h5, W1.reshape(kd, lane, h1), b1.reshape(1, h1), W2, b2.reshape(1, h2),
      W3, b3.reshape(1, ncls))


def kernel(x, table, W1, b1, W2, b2, W3, b3):
    batch, win = x.shape
    xi = x.astype(jnp.int32)
    x_pad = jnp.pad(xi, ((0, 0), (0, 128 - win)))
    table_sc = _tc_compact(table).reshape(table.shape)
    h = _sc_gather(table_sc, x_pad, win, table.shape[0])
    h5 = h.reshape(win // 4, batch, h.shape[-1])
    return _tc_mlp(h5, W1, b1, W2, b2, W3, b3)


# speedup vs baseline: 1.1410x; 1.1410x over previous
import jax
import jax.numpy as jnp
from jax.experimental import pallas as pl
from jax.experimental.pallas import tpu as pltpu
from jax.experimental.pallas import tpu_sc as plsc

_MLP_TILE = 1024
_NC, _NS = 2, 16
_NW = _NC * _NS
_SB = 128


def _sc_gather(table, x_pad, win):
    batch, win_pad = x_pad.shape
    emb = table.shape[1]
    kd = win // 4
    assert win % 4 == 0 and 4 * emb == 128
    npc = _SB * win
    assert npc % 128 == 0
    streams = npc // 128
    kblk = npc // kd
    b_per_w = batch // _NW
    chunks = b_per_w // _SB
    mesh = plsc.VectorSubcoreMesh(core_axis_name="core", subcore_axis_name="subcore")

    @pl.kernel(
        out_type=jax.ShapeDtypeStruct((kd * batch, 4 * emb), table.dtype),
        mesh=mesh,
        scratch_types=[
            pltpu.VMEM((_SB, win_pad), jnp.int32),
            pltpu.VMEM((npc,), jnp.int32),
            pltpu.VMEM((npc, emb), table.dtype),
            pltpu.SemaphoreType.DMA,
        ],
        compiler_params=pltpu.CompilerParams(
            use_tc_tiling_on_sc=False, needs_layout_passes=False
        ),
    )
    def gather_kernel(tab_hbm, i_hbm, o_hbm, idx_v, idxp_v, rows_v, sem):
        wid = jax.lax.axis_index("subcore") * _NC + jax.lax.axis_index("core")
        b0 = wid * b_per_w
        lane = jax.lax.broadcasted_iota(jnp.int32, (16,), 0)

        @pl.loop(0, chunks)
        def _(c):
            b = b0 + c * _SB
            pltpu.sync_copy(i_hbm.at[pl.ds(b, _SB)], idx_v)
            @pl.loop(0, win)
            def _(w):
                for u in range(_SB // 16):
                    rows = 16 * u + lane
                    cols = jnp.full((16,), 0, jnp.int32) + w
                    vals = plsc.load_gather(idx_v, [rows, cols])
                    idxp_v[pl.ds(w * _SB + 16 * u, 16)] = (
                        (vals & (-4 * _CQ))
                        + ((vals & (_CQ - 1)) << 2)
                        + ((vals >> _CQLOG) & 3)
                    )
            copies = [
                pltpu.async_copy(
                    tab_hbm.at[idxp_v.at[pl.ds(j * 128, 128)]],
                    rows_v.at[pl.ds(j * 128, 128)],
                    sem,
                )
                for j in range(streams)
            ]
            for cp in copies:
                cp.wait()
            for w in range(win):
                k, c4 = divmod(w, 4)
                pltpu.sync_copy(
                    rows_v.at[pl.ds(w * _SB, _SB)],
                    o_hbm.at[pl.ds(k * batch + b, _SB), pl.ds(c4 * emb, emb)],
                )

    return gather_kernel(table, x_pad)


_CQ = 512
_CQLOG = 9


def _compact_body(t0_ref, t1_ref, t2_ref, t3_ref, o_ref):
    emb = t0_ref.shape[0]
    for c, t_ref in enumerate((t0_ref, t1_ref, t2_ref, t3_ref)):
        o_ref[:, pl.ds(c * emb, emb)] = t_ref[...].T


def _tc_compact(table):
    v, emb = table.shape
    grid = -(-v // (4 * _CQ))
    nvb = -(-v // _CQ)
    t_t = table.T
    spec = lambda c: pl.BlockSpec(
        (emb, _CQ), lambda i, c=c: (0, jnp.minimum(4 * i + c, nvb - 1))
    )
    return pl.pallas_call(
        _compact_body,
        grid=(grid,),
        in_specs=[spec(0), spec(1), spec(2), spec(3)],
        out_specs=pl.BlockSpec((_CQ, 4 * emb), lambda i: (i, 0)),
        out_shape=jax.ShapeDtypeStruct((_CQ * grid, 4 * emb), jnp.float32),
    )(t_t, t_t, t_t, t_t)


def _mlp_body(h_ref, w1_ref, b1_ref, w2_ref, b2_ref, w3_ref, b3_ref, o_ref):
    kd = h_ref.shape[0]
    z = b1_ref[...] + jnp.dot(h_ref[0], w1_ref[0], preferred_element_type=jnp.float32)
    for k in range(1, kd):
        z = z + jnp.dot(h_ref[k], w1_ref[k], preferred_element_type=jnp.float32)
    z = jnp.maximum(z, 0.0)
    z = jnp.dot(z, w2_ref[...], preferred_element_type=jnp.float32) + b2_ref[...]
    z = jnp.maximum(z, 0.0)
    o_ref[...] = jnp.dot(z, w3_ref[...], preferred_element_type=jnp.float32) + b3_ref[...]


def _tc_mlp(h5, W1, b1, W2, b2, W3, b3):
    kd, batch, lane = h5.shape
    h1, h2, ncls = W1.shape[1], W2.shape[1], W3.shape[1]
    tile = min(_MLP_TILE, batch)
    grid = (batch // tile,)
    full = lambda shape: pl.BlockSpec(shape, lambda i: (0,) * len(shape))
    return pl.pallas_call(
        _mlp_body,
        grid=grid,
        in_specs=[
            pl.BlockSpec((kd, tile, lane), lambda i: (0, i, 0)),
            full((kd, lane, h1)),
            full((1, h1)),
            full((h1, h2)),
            full((1, h2)),
            full((h2, ncls)),
            full((1, ncls)),
        ],
        out_specs=pl.BlockSpec((tile, ncls), lambda i: (i, 0)),
        out_shape=jax.ShapeDtypeStruct((batch, ncls), jnp.float32),
    )(h5, W1.reshape(kd, lane, h1), b1.reshape(1, h1), W2, b2.reshape(1, h2),
      W3, b3.reshape(1, ncls))


def kernel(x, table, W1, b1, W2, b2, W3, b3):
    batch, win = x.shape
    xi = x.astype(jnp.int32)
    x_pad = jnp.pad(xi, ((0, 0), (0, 128 - win)))
    table_sc = _tc_compact(table).reshape(-1, table.shape[1])
    h = _sc_gather(table_sc, x_pad, win)
    h5 = h.reshape(win // 4, batch, h.shape[-1])
    return _tc_mlp(h5, W1, b1, W2, b2, W3, b3)


# speedup vs baseline: 1.4696x; 1.2880x over previous
import jax
import jax.numpy as jnp
from jax.experimental import pallas as pl
from jax.experimental.pallas import tpu as pltpu
from jax.experimental.pallas import tpu_sc as plsc

_MLP_TILE = 1024
_NC, _NS = 2, 16
_NW = _NC * _NS
_SB = 128


def _sc_gather(table, x_pad, win):
    batch, win_pad = x_pad.shape
    emb = table.shape[1]
    kd = win // 4
    assert win % 4 == 0 and 4 * emb == 128
    npc = _SB * win
    assert npc % 128 == 0
    streams = npc // 128
    kblk = npc // kd
    b_per_w = batch // _NW
    chunks = b_per_w // _SB
    mesh = plsc.VectorSubcoreMesh(core_axis_name="core", subcore_axis_name="subcore")

    @pl.kernel(
        out_type=jax.ShapeDtypeStruct((kd * batch, 4 * emb), table.dtype),
        mesh=mesh,
        scratch_types=[
            pltpu.VMEM((_SB, win_pad), jnp.int32),
            pltpu.VMEM((npc,), jnp.int32),
            pltpu.VMEM((npc, emb), table.dtype),
            pltpu.SemaphoreType.DMA,
        ],
        compiler_params=pltpu.CompilerParams(
            use_tc_tiling_on_sc=False, needs_layout_passes=False
        ),
    )
    def gather_kernel(tab_hbm, i_hbm, o_hbm, idx_v, idxp_v, rows_v, sem):
        wid = jax.lax.axis_index("subcore") * _NC + jax.lax.axis_index("core")
        b0 = wid * b_per_w
        lane = jax.lax.broadcasted_iota(jnp.int32, (16,), 0)

        @pl.loop(0, chunks)
        def _(c):
            b = b0 + c * _SB
            pltpu.sync_copy(i_hbm.at[pl.ds(b, _SB)], idx_v)
            @pl.loop(0, win)
            def _(w):
                for u in range(_SB // 16):
                    rows = 16 * u + lane
                    cols = jnp.full((16,), 0, jnp.int32) + w
                    vals = plsc.load_gather(idx_v, [rows, cols])
                    idxp_v[pl.ds(w * _SB + 16 * u, 16)] = (
                        (vals & (-4 * _CQ))
                        + ((vals & (_CQ - 1)) << 2)
                        + ((vals >> _CQLOG) & 3)
                    )
            copies = [
                pltpu.async_copy(
                    tab_hbm.at[idxp_v.at[pl.ds(j * 128, 128)]],
                    rows_v.at[pl.ds(j * 128, 128)],
                    sem,
                )
                for j in range(streams)
            ]
            for cp in copies:
                cp.wait()
            for w in range(win):
                k, c4 = divmod(w, 4)
                pltpu.sync_copy(
                    rows_v.at[pl.ds(w * _SB, _SB)],
                    o_hbm.at[pl.ds(k * batch + b, _SB), pl.ds(c4 * emb, emb)],
                )

    return gather_kernel(table, x_pad)


_CQ = 1024
_CQLOG = 10


def _compact_body(t0_ref, t1_ref, t2_ref, t3_ref, o_ref):
    emb = t0_ref.shape[0]
    rng = jax.lax.broadcasted_iota(jnp.int32, (emb, emb), 0)
    eye = (rng == rng.T).astype(jnp.float32)
    for c, t_ref in enumerate((t0_ref, t1_ref, t2_ref, t3_ref)):
        o_ref[:, pl.ds(c * emb, emb)] = jax.lax.dot_general(
            t_ref[...], eye, (((0,), (0,)), ((), ())),
            preferred_element_type=jnp.float32,
        )


def _tc_compact(table):
    v, emb = table.shape
    grid = -(-v // (4 * _CQ))
    nvb = -(-v // _CQ)
    t_t = table.T
    spec = lambda c: pl.BlockSpec(
        (emb, _CQ), lambda i, c=c: (0, jnp.minimum(4 * i + c, nvb - 1))
    )
    return pl.pallas_call(
        _compact_body,
        grid=(grid,),
        in_specs=[spec(0), spec(1), spec(2), spec(3)],
        out_specs=pl.BlockSpec((_CQ, 4 * emb), lambda i: (i, 0)),
        out_shape=jax.ShapeDtypeStruct((_CQ * grid, 4 * emb), jnp.float32),
    )(t_t, t_t, t_t, t_t)


def _mlp_body(h_ref, w1_ref, b1_ref, w2_ref, b2_ref, w3_ref, b3_ref, o_ref):
    kd = h_ref.shape[0]
    z = b1_ref[...] + jnp.dot(h_ref[0], w1_ref[0], preferred_element_type=jnp.float32)
    for k in range(1, kd):
        z = z + jnp.dot(h_ref[k], w1_ref[k], preferred_element_type=jnp.float32)
    z = jnp.maximum(z, 0.0)
    z = jnp.dot(z, w2_ref[...], preferred_element_type=jnp.float32) + b2_ref[...]
    z = jnp.maximum(z, 0.0)
    o_ref[...] = jnp.dot(z, w3_ref[...], preferred_element_type=jnp.float32) + b3_ref[...]


def _tc_mlp(h5, W1, b1, W2, b2, W3, b3):
    kd, batch, lane = h5.shape
    h1, h2, ncls = W1.shape[1], W2.shape[1], W3.shape[1]
    tile = min(_MLP_TILE, batch)
    grid = (batch // tile,)
    full = lambda shape: pl.BlockSpec(shape, lambda i: (0,) * len(shape))
    return pl.pallas_call(
        _mlp_body,
        grid=grid,
        in_specs=[
            pl.BlockSpec((kd, tile, lane), lambda i: (0, i, 0)),
            full((kd, lane, h1)),
            full((1, h1)),
            full((h1, h2)),
            full((1, h2)),
            full((h2, ncls)),
            full((1, ncls)),
        ],
        out_specs=pl.BlockSpec((tile, ncls), lambda i: (i, 0)),
        out_shape=jax.ShapeDtypeStruct((batch, ncls), jnp.float32),
    )(h5, W1.reshape(kd, lane, h1), b1.reshape(1, h1), W2, b2.reshape(1, h2),
      W3, b3.reshape(1, ncls))


def kernel(x, table, W1, b1, W2, b2, W3, b3):
    batch, win = x.shape
    xi = x.astype(jnp.int32)
    x_pad = jnp.pad(xi, ((0, 0), (0, 128 - win)))
    table_sc = _tc_compact(table).reshape(-1, table.shape[1])
    h = _sc_gather(table_sc, x_pad, win)
    h5 = h.reshape(win // 4, batch, h.shape[-1])
    return _tc_mlp(h5, W1, b1, W2, b2, W3, b3)


# speedup vs baseline: 1.6287x; 1.1082x over previous
import jax
import jax.numpy as jnp
from jax.experimental import pallas as pl
from jax.experimental.pallas import tpu as pltpu
from jax.experimental.pallas import tpu_sc as plsc

_MLP_TILE = 1024
_NC, _NS = 2, 16
_NW = _NC * _NS
_SB = 128


def _sc_gather(table, x_pad, win):
    batch, win_pad = x_pad.shape
    emb = table.shape[1]
    kd = win // 4
    assert win % 4 == 0 and 4 * emb == 128
    npc = _SB * win
    assert npc % 128 == 0
    streams = npc // 128
    kblk = npc // kd
    b_per_w = batch // _NW
    chunks = b_per_w // _SB
    mesh = plsc.VectorSubcoreMesh(core_axis_name="core", subcore_axis_name="subcore")

    @pl.kernel(
        out_type=jax.ShapeDtypeStruct((kd * batch, 4 * emb), table.dtype),
        mesh=mesh,
        scratch_types=[
            pltpu.VMEM((_SB, win_pad), jnp.int32),
            pltpu.VMEM((npc,), jnp.int32),
            pltpu.VMEM((npc, emb), table.dtype),
            pltpu.SemaphoreType.DMA,
        ],
        compiler_params=pltpu.CompilerParams(
            use_tc_tiling_on_sc=False, needs_layout_passes=False
        ),
    )
    def gather_kernel(tab_hbm, i_hbm, o_hbm, idx_v, idxp_v, rows_v, sem):
        wid = jax.lax.axis_index("subcore") * _NC + jax.lax.axis_index("core")
        b0 = wid * b_per_w
        lane = jax.lax.broadcasted_iota(jnp.int32, (16,), 0)

        @pl.loop(0, chunks)
        def _(c):
            b = b0 + c * _SB
            pltpu.sync_copy(i_hbm.at[pl.ds(b, _SB)], idx_v)
            @pl.loop(0, win)
            def _(w):
                for u in range(_SB // 16):
                    rows = 16 * u + lane
                    cols = jnp.full((16,), 0, jnp.int32) + w
                    vals = plsc.load_gather(idx_v, [rows, cols])
                    idxp_v[pl.ds(w * _SB + 16 * u, 16)] = (
                        (vals & (-4 * _CQ))
                        + ((vals & (_CQ - 1)) << 2)
                        + ((vals >> _CQLOG) & 3)
                    )
            copies = [
                pltpu.async_copy(
                    tab_hbm.at[idxp_v.at[pl.ds(j * 128, 128)]],
                    rows_v.at[pl.ds(j * 128, 128)],
                    sem,
                )
                for j in range(streams)
            ]
            for cp in copies:
                cp.wait()
            for w in range(win):
                k, c4 = divmod(w, 4)
                pltpu.sync_copy(
                    rows_v.at[pl.ds(w * _SB, _SB)],
                    o_hbm.at[pl.ds(k * batch + b, _SB), pl.ds(c4 * emb, emb)],
                )

    return gather_kernel(table, x_pad)


_CQ = 2048
_CQLOG = 11


def _compact_body(t0_ref, t1_ref, t2_ref, t3_ref, o_ref):
    emb = t0_ref.shape[0]
    rng = jax.lax.broadcasted_iota(jnp.int32, (emb, emb), 0)
    eye = (rng == rng.T).astype(jnp.float32)
    for c, t_ref in enumerate((t0_ref, t1_ref, t2_ref, t3_ref)):
        o_ref[:, pl.ds(c * emb, emb)] = jax.lax.dot_general(
            t_ref[...], eye, (((0,), (0,)), ((), ())),
            preferred_element_type=jnp.float32,
        )


def _tc_compact(table):
    v, emb = table.shape
    grid = -(-v // (4 * _CQ))
    nvb = -(-v // _CQ)
    t_t = table.T
    spec = lambda c: pl.BlockSpec(
        (emb, _CQ), lambda i, c=c: (0, jnp.minimum(4 * i + c, nvb - 1))
    )
    return pl.pallas_call(
        _compact_body,
        grid=(grid,),
        in_specs=[spec(0), spec(1), spec(2), spec(3)],
        out_specs=pl.BlockSpec((_CQ, 4 * emb), lambda i: (i, 0)),
        out_shape=jax.ShapeDtypeStruct((_CQ * grid, 4 * emb), jnp.float32),
    )(t_t, t_t, t_t, t_t)


def _mlp_body(h_ref, w1_ref, b1_ref, w2_ref, b2_ref, w3_ref, b3_ref, o_ref):
    kd = h_ref.shape[0]
    z = b1_ref[...] + jnp.dot(h_ref[0], w1_ref[0], preferred_element_type=jnp.float32)
    for k in range(1, kd):
        z = z + jnp.dot(h_ref[k], w1_ref[k], preferred_element_type=jnp.float32)
    z = jnp.maximum(z, 0.0)
    z = jnp.dot(z, w2_ref[...], preferred_element_type=jnp.float32) + b2_ref[...]
    z = jnp.maximum(z, 0.0)
    o_ref[...] = jnp.dot(z, w3_ref[...], preferred_element_type=jnp.float32) + b3_ref[...]


def _tc_mlp(h5, W1, b1, W2, b2, W3, b3):
    kd, batch, lane = h5.shape
    h1, h2, ncls = W1.shape[1], W2.shape[1], W3.shape[1]
    tile = min(_MLP_TILE, batch)
    grid = (batch // tile,)
    full = lambda shape: pl.BlockSpec(shape, lambda i: (0,) * len(shape))
    return pl.pallas_call(
        _mlp_body,
        grid=grid,
        in_specs=[
            pl.BlockSpec((kd, tile, lane), lambda i: (0, i, 0)),
            full((kd, lane, h1)),
            full((1, h1)),
            full((h1, h2)),
            full((1, h2)),
            full((h2, ncls)),
            full((1, ncls)),
        ],
        out_specs=pl.BlockSpec((tile, ncls), lambda i: (i, 0)),
        out_shape=jax.ShapeDtypeStruct((batch, ncls), jnp.float32),
    )(h5, W1.reshape(kd, lane, h1), b1.reshape(1, h1), W2, b2.reshape(1, h2),
      W3, b3.reshape(1, ncls))


def kernel(x, table, W1, b1, W2, b2, W3, b3):
    batch, win = x.shape
    xi = x.astype(jnp.int32)
    x_pad = jnp.pad(xi, ((0, 0), (0, 128 - win)))
    table_sc = _tc_compact(table).reshape(-1, table.shape[1])
    h = _sc_gather(table_sc, x_pad, win)
    h5 = h.reshape(win // 4, batch, h.shape[-1])
    return _tc_mlp(h5, W1, b1, W2, b2, W3, b3)


# speedup vs baseline: 1.6648x; 1.0222x over previous
import jax
import jax.numpy as jnp
from jax.experimental import pallas as pl
from jax.experimental.pallas import tpu as pltpu
from jax.experimental.pallas import tpu_sc as plsc

_MLP_TILE = 1024
_NC, _NS = 2, 16
_NW = _NC * _NS
_SB = 128


def _sc_gather(table, x_pad, win):
    batch, win_pad = x_pad.shape
    emb = table.shape[1]
    kd = win // 4
    assert win % 4 == 0 and 4 * emb == 128
    npc = _SB * win
    assert npc % 128 == 0
    streams = npc // 128
    kblk = npc // kd
    b_per_w = batch // _NW
    chunks = b_per_w // _SB
    mesh = plsc.VectorSubcoreMesh(core_axis_name="core", subcore_axis_name="subcore")

    @pl.kernel(
        out_type=jax.ShapeDtypeStruct((kd * batch, 4 * emb), table.dtype),
        mesh=mesh,
        scratch_types=[
            pltpu.VMEM((_SB, win_pad), jnp.int32),
            pltpu.VMEM((npc,), jnp.int32),
            pltpu.VMEM((npc, emb), table.dtype),
            pltpu.SemaphoreType.DMA,
        ],
        compiler_params=pltpu.CompilerParams(
            use_tc_tiling_on_sc=False, needs_layout_passes=False
        ),
    )
    def gather_kernel(tab_hbm, i_hbm, o_hbm, idx_v, idxp_v, rows_v, sem):
        wid = jax.lax.axis_index("subcore") * _NC + jax.lax.axis_index("core")
        b0 = wid * b_per_w
        lane = jax.lax.broadcasted_iota(jnp.int32, (16,), 0)

        @pl.loop(0, chunks)
        def _(c):
            b = b0 + c * _SB
            pltpu.sync_copy(i_hbm.at[pl.ds(b, _SB)], idx_v)
            @pl.loop(0, win)
            def _(w):
                for u in range(_SB // 16):
                    rows = 16 * u + lane
                    cols = jnp.full((16,), 0, jnp.int32) + w
                    vals = plsc.load_gather(idx_v, [rows, cols])
                    idxp_v[pl.ds(w * _SB + 16 * u, 16)] = (
                        (vals & (-4 * _CQ))
                        + ((vals & (_CQ - 1)) << 2)
                        + ((vals >> _CQLOG) & 3)
                    )
            copies = [
                pltpu.async_copy(
                    tab_hbm.at[idxp_v.at[pl.ds(j * 128, 128)]],
                    rows_v.at[pl.ds(j * 128, 128)],
                    sem,
                )
                for j in range(streams)
            ]
            for cp in copies:
                cp.wait()
            for w in range(win):
                k, c4 = divmod(w, 4)
                pltpu.sync_copy(
                    rows_v.at[pl.ds(w * _SB, _SB)],
                    o_hbm.at[pl.ds(k * batch + b, _SB), pl.ds(c4 * emb, emb)],
                )

    return gather_kernel(table, x_pad)


_CQ = 4096
_CQLOG = 12


def _compact_body(t0_ref, t1_ref, t2_ref, t3_ref, o_ref):
    emb = t0_ref.shape[0]
    rng = jax.lax.broadcasted_iota(jnp.int32, (emb, emb), 0)
    eye = (rng == rng.T).astype(jnp.float32)
    for c, t_ref in enumerate((t0_ref, t1_ref, t2_ref, t3_ref)):
        o_ref[:, pl.ds(c * emb, emb)] = jax.lax.dot_general(
            t_ref[...], eye, (((0,), (0,)), ((), ())),
            preferred_element_type=jnp.float32,
        )


def _tc_compact(table):
    v, emb = table.shape
    grid = -(-v // (4 * _CQ))
    nvb = -(-v // _CQ)
    t_t = table.T
    spec = lambda c: pl.BlockSpec(
        (emb, _CQ), lambda i, c=c: (0, jnp.minimum(4 * i + c, nvb - 1))
    )
    return pl.pallas_call(
        _compact_body,
        grid=(grid,),
        in_specs=[spec(0), spec(1), spec(2), spec(3)],
        out_specs=pl.BlockSpec((_CQ, 4 * emb), lambda i: (i, 0)),
        out_shape=jax.ShapeDtypeStruct((_CQ * grid, 4 * emb), jnp.float32),
    )(t_t, t_t, t_t, t_t)


def _mlp_body(h_ref, w1_ref, b1_ref, w2_ref, b2_ref, w3_ref, b3_ref, o_ref):
    kd = h_ref.shape[0]
    z = b1_ref[...] + jnp.dot(h_ref[0], w1_ref[0], preferred_element_type=jnp.float32)
    for k in range(1, kd):
        z = z + jnp.dot(h_ref[k], w1_ref[k], preferred_element_type=jnp.float32)
    z = jnp.maximum(z, 0.0)
    z = jnp.dot(z, w2_ref[...], preferred_element_type=jnp.float32) + b2_ref[...]
    z = jnp.maximum(z, 0.0)
    o_ref[...] = jnp.dot(z, w3_ref[...], preferred_element_type=jnp.float32) + b3_ref[...]


def _tc_mlp(h5, W1, b1, W2, b2, W3, b3):
    kd, batch, lane = h5.shape
    h1, h2, ncls = W1.shape[1], W2.shape[1], W3.shape[1]
    tile = min(_MLP_TILE, batch)
    grid = (batch // tile,)
    full = lambda shape: pl.BlockSpec(shape, lambda i: (0,) * len(shape))
    return pl.pallas_call(
        _mlp_body,
        grid=grid,
        in_specs=[
            pl.BlockSpec((kd, tile, lane), lambda i: (0, i, 0)),
            full((kd, lane, h1)),
            full((1, h1)),
            full((h1, h2)),
            full((1, h2)),
            full((h2, ncls)),
            full((1, ncls)),
        ],
        out_specs=pl.BlockSpec((tile, ncls), lambda i: (i, 0)),
        out_shape=jax.ShapeDtypeStruct((batch, ncls), jnp.float32),
    )(h5, W1.reshape(kd, lane, h1), b1.reshape(1, h1), W2, b2.reshape(1, h2),
      W3, b3.reshape(1, ncls))


def kernel(x, table, W1, b1, W2, b2, W3, b3):
    batch, win = x.shape
    xi = x.astype(jnp.int32)
    x_pad = jnp.pad(xi, ((0, 0), (0, 128 - win)))
    table_sc = _tc_compact(table).reshape(-1, table.shape[1])
    h = _sc_gather(table_sc, x_pad, win)
    h5 = h.reshape(win // 4, batch, h.shape[-1])
    return _tc_mlp(h5, W1, b1, W2, b2, W3, b3)
